# baseline (speedup 1.0000x reference)
import functools

import jax
import jax.numpy as jnp
from jax import lax
from jax.experimental import pallas as pl
from jax.experimental.pallas import tpu as pltpu
from jax.experimental.pallas import tpu_sc as plsc

NC, NS, L = 2, 16, 16
D = 128
CH = 128
BLK = 8
C1 = 80
C2 = 48
ACC_ROWS = 10240
PAIR_ROWS = 6144


def _spmm_body(pre_hbm, s1, d1, w1, s2, d2, w2, pmain, pout,
               srcv, dstv, wv, rows0, rows1, acc,
               gsem0, gsem1, ssem0, ssem1):
    cid = lax.axis_index("c")
    sid = lax.axis_index("s")
    row0 = sid * (ACC_ROWS // NS)

    def zero_rows_buf():
        def zb(i, _):
            for f in range(D // L):
                rows0[i, pl.ds(f * L, L)] = jnp.zeros((L,), jnp.float32)
            return 0
        lax.fori_loop(0, CH, zb, 0)

    def zero_acc_slice(r0):
        for k in range(ACC_ROWS // NS // CH):
            pltpu.sync_copy(rows0, acc.at[pl.ds(r0 + k * CH, CH)])

    zero_rows_buf()
    zero_acc_slice(row0)

    def scale(buf, j):
        def grp(g, _):
            wvec = wv[j, pl.ds(g * L, L)]
            for l in range(L):
                w = wvec[l]
                i = g * L + l
                for f in range(D // L):
                    sl = pl.ds(f * L, L)
                    buf[i, sl] = buf[i, sl] * w
            return 0
        lax.fori_loop(0, CH // L, grp, 0)

    def run_phase(sref, dref, wref, nblocks):
        bufs = (rows0, rows1)
        gsems = (gsem0, gsem1)
        ssems = (ssem0, ssem1)

        def blk_body(b, _):
            pltpu.sync_copy(sref.at[cid, sid, pl.ds(b * BLK, BLK)], srcv)
            pltpu.sync_copy(dref.at[cid, sid, pl.ds(b * BLK, BLK)], dstv)
            pltpu.sync_copy(wref.at[cid, sid, pl.ds(b * BLK, BLK)], wv)

            gd = [None] * BLK
            sd = [None] * BLK
            gd[0] = pltpu.async_copy(pre_hbm.at[srcv.at[0]], bufs[0],
                                     gsems[0])
            for k in range(BLK):
                p = k % 2
                if k + 1 < BLK:
                    if k >= 1:
                        sd[k - 1].wait()
                    gd[k + 1] = pltpu.async_copy(
                        pre_hbm.at[srcv.at[k + 1]], bufs[1 - p],
                        gsems[1 - p])
                gd[k].wait()
                scale(bufs[p], k)
                sd[k] = pltpu.async_copy(bufs[p], acc.at[dstv.at[k]],
                                         ssems[p], add=True)
            sd[BLK - 2].wait()
            sd[BLK - 1].wait()
            return 0
        lax.fori_loop(0, nblocks, blk_body, 0)

    plsc.subcore_barrier()
    run_phase(s1, d1, w1, C1 // BLK)
    plsc.subcore_barrier()

    for k in range(ACC_ROWS // NS // CH):
        r = row0 + k * CH
        pltpu.sync_copy(acc.at[pl.ds(r, CH)], rows0)
        pltpu.sync_copy(rows0, pmain.at[cid, pl.ds(r, CH)])
    zero_rows_buf()
    zero_acc_slice(row0)

    plsc.subcore_barrier()
    run_phase(s2, d2, w2, C2 // BLK)
    plsc.subcore_barrier()

    prow0 = sid * (PAIR_ROWS // NS)
    for k in range(PAIR_ROWS // NS // CH):
        r = prow0 + k * CH
        pltpu.sync_copy(acc.at[pl.ds(r, CH)], rows0)
        pltpu.sync_copy(rows0, pout.at[cid, pl.ds(r, CH)])


_spmm = functools.partial(
    pl.kernel,
    out_type=(jax.ShapeDtypeStruct((NC, ACC_ROWS, D), jnp.float32),
              jax.ShapeDtypeStruct((NC, PAIR_ROWS, D), jnp.float32)),
    mesh=plsc.VectorSubcoreMesh(core_axis_name="c", subcore_axis_name="s"),
    scratch_types=[
        pltpu.VMEM((BLK, CH), jnp.int32),
        pltpu.VMEM((BLK, CH), jnp.int32),
        pltpu.VMEM((BLK, CH), jnp.float32),
        pltpu.VMEM((CH, D), jnp.float32),
        pltpu.VMEM((CH, D), jnp.float32),
        pltpu.VMEM_SHARED((ACC_ROWS, D), jnp.float32),
        pltpu.SemaphoreType.DMA,
        pltpu.SemaphoreType.DMA,
        pltpu.SemaphoreType.DMA,
        pltpu.SemaphoreType.DMA,
    ],
)(_spmm_body)


def _dense_body(pre_ref, a0_ref, a1_ref, pair_ref, q_ref, w1_ref, w2_ref,
                b_ref, m_ref, o_ref):
    hi = lax.Precision.HIGHEST
    agg = a0_ref[...] + a1_ref[...]
    h = jnp.tanh(lax.dot(agg, q_ref[...], precision=hi))
    z = (lax.dot(pre_ref[...], w1_ref[...], precision=hi)
         + lax.dot(h, w2_ref[...], precision=hi) + b_ref[...])
    z = jnp.where(z >= 0, z, 0.01 * z)
    z = z / (jnp.sqrt(jnp.sum(z * z, axis=1, keepdims=True)) + 1e-12)
    o_ref[...] = z + jnp.tanh(lax.dot(pair_ref[...], m_ref[...], precision=hi))


def _dense(pre, a0, a1, pair, Q, W1, W2, b, M, blk):
    n = pre.shape[0]
    grid = n // blk
    row = pl.BlockSpec((blk, D), lambda i: (i, 0))
    full = pl.BlockSpec((D, D), lambda i: (0, 0))
    bias = pl.BlockSpec((1, D), lambda i: (0, 0))
    return pl.pallas_call(
        _dense_body,
        grid=(grid,),
        in_specs=[row, row, row, row, full, full, full, bias, full],
        out_specs=row,
        out_shape=jax.ShapeDtypeStruct((n, D), jnp.float32),
    )(pre, a0, a1, pair, Q, W1, W2, b, M)


def _pad_to(x, total):
    return jnp.concatenate(
        [x, jnp.zeros((total - x.shape[0],), dtype=x.dtype)])


def _pad_idx(x, total, base, span):
    npad = total - x.shape[0]
    pad = base + (jnp.arange(npad, dtype=x.dtype) % span)
    return jnp.concatenate([x, pad])


def kernel(edge_index, edge_weight, sym_pair_edge_index, sym_pair_weight,
           herb_pair_edge_index, herb_pair_weight, user_embedding,
           item_embedding, Q_user_0, W_gc_user_0, b_gc_user_0, Q_item_0,
           W_gc_item_0, b_gc_item_0, M_user, M_item):
    nu = user_embedding.shape[0]
    ni = item_embedding.shape[0]
    pre = jnp.concatenate([user_embedding, item_embedding], axis=0)

    n = nu + ni
    t1 = NC * NS * C1 * CH
    s1 = _pad_idx(edge_index[0], t1, 0, n).reshape(NC, NS, C1, CH)
    d1 = _pad_idx(edge_index[1], t1, n, ACC_ROWS - n).reshape(NC, NS, C1, CH)
    w1 = _pad_to(edge_weight, t1).reshape(NC, NS, C1, CH)

    t2 = NS * C2 * CH
    us = _pad_idx(sym_pair_edge_index[0], t2, 0, nu).reshape(NS, C2, CH)
    ud = _pad_idx(sym_pair_edge_index[1], t2, nu,
                  PAIR_ROWS - nu).reshape(NS, C2, CH)
    uw = _pad_to(sym_pair_weight, t2).reshape(NS, C2, CH)
    isrc = _pad_idx(herb_pair_edge_index[0] + nu, t2, nu, ni).reshape(
        NS, C2, CH)
    idst = _pad_idx(herb_pair_edge_index[1], t2, ni,
                    PAIR_ROWS - ni).reshape(NS, C2, CH)
    iw = _pad_to(herb_pair_weight, t2).reshape(NS, C2, CH)
    s2 = jnp.stack([us, isrc])
    d2 = jnp.stack([ud, idst])
    w2 = jnp.stack([uw, iw])

    pmain, pout = _spmm(pre, s1, d1, w1, s2, d2, w2)

    u_out = _dense(user_embedding, pmain[0, :nu], pmain[1, :nu],
                   pout[0, :nu], Q_user_0, W_gc_user_0[:D], W_gc_user_0[D:],
                   b_gc_user_0, M_user, blk=1000)
    i_out = _dense(item_embedding, pmain[0, nu:nu + ni],
                   pmain[1, nu:nu + ni], pout[1, :ni], Q_item_0,
                   W_gc_item_0[:D], W_gc_item_0[D:], b_gc_item_0, M_item,
                   blk=1000)
    return jnp.concatenate([u_out, i_out], axis=0)


# speedup vs baseline: 1.0505x; 1.0505x over previous
import functools

import jax
import jax.numpy as jnp
from jax import lax
from jax.experimental import pallas as pl
from jax.experimental.pallas import tpu as pltpu
from jax.experimental.pallas import tpu_sc as plsc

NC, NS, L = 2, 16, 16
D = 128
CH = 128
BLK = 8
C1 = 80
C2 = 40
ACC_ROWS = 10240
PAIR_OFF = 6144
ZCH = 128


def _spmm_body(pre_hbm, s1, d1, w1, s2, d2, w2, pmain, pout,
               srcv, dstv, wv, rows0, rows1, acc,
               gsem0, gsem1, ssem0, ssem1):
    cid = lax.axis_index("c")
    sid = lax.axis_index("s")
    row0 = sid * (ACC_ROWS // NS)

    def zb(i, _):
        for f in range(D // L):
            rows0[i, pl.ds(f * L, L)] = jnp.zeros((L,), jnp.float32)
        return 0
    lax.fori_loop(0, CH, zb, 0)
    for k in range(ACC_ROWS // NS // ZCH):
        pltpu.sync_copy(rows0, acc.at[pl.ds(row0 + k * ZCH, ZCH)])

    def scale(buf, j):
        def grp(g, _):
            wvec = wv[j, pl.ds(g * L, L)]
            for l in range(L):
                w = wvec[l]
                i = g * L + l
                for f in range(D // L):
                    sl = pl.ds(f * L, L)
                    buf[i, sl] = buf[i, sl] * w
            return 0
        lax.fori_loop(0, CH // L, grp, 0)

    def run_phase(sref, dref, wref, nblocks):
        bufs = (rows0, rows1)
        gsems = (gsem0, gsem1)
        ssems = (ssem0, ssem1)

        def blk_body(b, _):
            pltpu.sync_copy(sref.at[cid, sid, pl.ds(b * BLK, BLK)], srcv)
            pltpu.sync_copy(dref.at[cid, sid, pl.ds(b * BLK, BLK)], dstv)
            pltpu.sync_copy(wref.at[cid, sid, pl.ds(b * BLK, BLK)], wv)

            gd = [None] * BLK
            sd = [None] * BLK
            gd[0] = pltpu.async_copy(pre_hbm.at[srcv.at[0]], bufs[0],
                                     gsems[0])
            for k in range(BLK):
                p = k % 2
                gd[k].wait()
                if k + 1 < BLK:
                    if k >= 1:
                        sd[k - 1].wait()
                    gd[k + 1] = pltpu.async_copy(
                        pre_hbm.at[srcv.at[k + 1]], bufs[1 - p],
                        gsems[1 - p])
                scale(bufs[p], k)
                sd[k] = pltpu.async_copy(bufs[p], acc.at[dstv.at[k]],
                                         ssems[p], add=True)
            sd[BLK - 2].wait()
            sd[BLK - 1].wait()
            return 0
        lax.fori_loop(0, nblocks, blk_body, 0)

    plsc.subcore_barrier()
    run_phase(s1, d1, w1, C1 // BLK)
    plsc.subcore_barrier()

    for k in range(ACC_ROWS // NS // ZCH):
        r = row0 + k * ZCH
        pltpu.sync_copy(acc.at[pl.ds(r, ZCH)], rows0)
        pltpu.sync_copy(rows0, pmain.at[cid, pl.ds(r, ZCH)])

    plsc.subcore_barrier()
    run_phase(s2, d2, w2, C2 // BLK)
    plsc.subcore_barrier()

    for k in range(ACC_ROWS // NS // ZCH):
        r = row0 + k * ZCH
        pltpu.sync_copy(acc.at[pl.ds(r, ZCH)], rows0)
        pltpu.sync_copy(rows0, pout.at[cid, pl.ds(r, ZCH)])


_spmm = functools.partial(
    pl.kernel,
    out_type=(jax.ShapeDtypeStruct((NC, ACC_ROWS, D), jnp.float32),
              jax.ShapeDtypeStruct((NC, ACC_ROWS, D), jnp.float32)),
    mesh=plsc.VectorSubcoreMesh(core_axis_name="c", subcore_axis_name="s"),
    scratch_types=[
        pltpu.VMEM((BLK, CH), jnp.int32),
        pltpu.VMEM((BLK, CH), jnp.int32),
        pltpu.VMEM((BLK, CH), jnp.float32),
        pltpu.VMEM((CH, D), jnp.float32),
        pltpu.VMEM((CH, D), jnp.float32),
        pltpu.VMEM_SHARED((ACC_ROWS, D), jnp.float32),
        pltpu.SemaphoreType.DMA,
        pltpu.SemaphoreType.DMA,
        pltpu.SemaphoreType.DMA,
        pltpu.SemaphoreType.DMA,
    ],
)(_spmm_body)


def _dense_body(pre_ref, a0_ref, a1_ref, m0_ref, m1_ref, p0_ref, p1_ref,
                q_ref, w1_ref, w2_ref, b_ref, m_ref, o_ref):
    hi = lax.Precision.HIGHEST
    agg = a0_ref[...] + a1_ref[...]
    pair = (p0_ref[...] - m0_ref[...]) + (p1_ref[...] - m1_ref[...])
    h = jnp.tanh(lax.dot(agg, q_ref[...], precision=hi))
    z = (lax.dot(pre_ref[...], w1_ref[...], precision=hi)
         + lax.dot(h, w2_ref[...], precision=hi) + b_ref[...])
    z = jnp.where(z >= 0, z, 0.01 * z)
    z = z / (jnp.sqrt(jnp.sum(z * z, axis=1, keepdims=True)) + 1e-12)
    o_ref[...] = z + jnp.tanh(lax.dot(pair, m_ref[...], precision=hi))


def _dense(pre, a0, a1, m0, m1, p0, p1, Q, W1, W2, b, M, blk):
    n = pre.shape[0]
    grid = n // blk
    row = pl.BlockSpec((blk, D), lambda i: (i, 0))
    full = pl.BlockSpec((D, D), lambda i: (0, 0))
    bias = pl.BlockSpec((1, D), lambda i: (0, 0))
    return pl.pallas_call(
        _dense_body,
        grid=(grid,),
        in_specs=[row, row, row, row, row, row, row, full, full, full,
                  bias, full],
        out_specs=row,
        out_shape=jax.ShapeDtypeStruct((n, D), jnp.float32),
    )(pre, a0, a1, m0, m1, p0, p1, Q, W1, W2, b, M)


def _pad_to(x, total):
    return jnp.concatenate(
        [x, jnp.zeros((total - x.shape[0],), dtype=x.dtype)])


def _pad_idx(x, total, base, span):
    npad = total - x.shape[0]
    pad = base + (jnp.arange(npad, dtype=x.dtype) % span)
    return jnp.concatenate([x, pad])


def kernel(edge_index, edge_weight, sym_pair_edge_index, sym_pair_weight,
           herb_pair_edge_index, herb_pair_weight, user_embedding,
           item_embedding, Q_user_0, W_gc_user_0, b_gc_user_0, Q_item_0,
           W_gc_item_0, b_gc_item_0, M_user, M_item):
    nu = user_embedding.shape[0]
    ni = item_embedding.shape[0]
    n = nu + ni
    pre = jnp.concatenate([user_embedding, item_embedding], axis=0)

    t1 = NC * NS * C1 * CH
    s1 = _pad_idx(edge_index[0], t1, 0, n).reshape(NC, NS, C1, CH)
    d1 = _pad_idx(edge_index[1], t1, n, ACC_ROWS - n).reshape(NC, NS, C1, CH)
    w1 = _pad_to(edge_weight, t1).reshape(NC, NS, C1, CH)

    t2 = NC * NS * C2 * CH
    ps = jnp.concatenate([sym_pair_edge_index[0],
                          herb_pair_edge_index[0] + nu])
    pd = jnp.concatenate([sym_pair_edge_index[1],
                          herb_pair_edge_index[1] + PAIR_OFF])
    pw = jnp.concatenate([sym_pair_weight, herb_pair_weight])
    s2 = _pad_idx(ps, t2, 0, n).reshape(NC, NS, C2, CH)
    d2 = _pad_idx(pd, t2, PAIR_OFF + ni,
                  ACC_ROWS - PAIR_OFF - ni).reshape(NC, NS, C2, CH)
    w2 = _pad_to(pw, t2).reshape(NC, NS, C2, CH)

    pmain, pout = _spmm(pre, s1, d1, w1, s2, d2, w2)

    u_out = _dense(user_embedding, pmain[0, :nu], pmain[1, :nu],
                   pmain[0, :nu], pmain[1, :nu],
                   pout[0, :nu], pout[1, :nu], Q_user_0, W_gc_user_0[:D],
                   W_gc_user_0[D:], b_gc_user_0, M_user, blk=1000)
    i_out = _dense(item_embedding, pmain[0, nu:n], pmain[1, nu:n],
                   pmain[0, PAIR_OFF:PAIR_OFF + ni],
                   pmain[1, PAIR_OFF:PAIR_OFF + ni],
                   pout[0, PAIR_OFF:PAIR_OFF + ni],
                   pout[1, PAIR_OFF:PAIR_OFF + ni], Q_item_0,
                   W_gc_item_0[:D], W_gc_item_0[D:], b_gc_item_0, M_item,
                   blk=1000)
    return jnp.concatenate([u_out, i_out], axis=0)


# speedup vs baseline: 1.1322x; 1.0778x over previous
import functools

import jax
import jax.numpy as jnp
from jax import lax
from jax.experimental import pallas as pl
from jax.experimental.pallas import tpu as pltpu
from jax.experimental.pallas import tpu_sc as plsc

NC, NS, L = 2, 16, 16
D = 128
CH = 128
BLK = 8
C1 = 80
C2 = 40
ACC_ROWS = 10240
PAIR_OFF = 6000
ZCH = 128


def _spmm_body(pre_hbm, s1, d1, w1, s2, d2, w2, pmain, pout,
               srcv, dstv, wv, rows0, rows1, acc,
               gsem0, gsem1, ssem0, ssem1):
    cid = lax.axis_index("c")
    sid = lax.axis_index("s")
    row0 = sid * (ACC_ROWS // NS)

    def zb(i, _):
        for f in range(D // L):
            rows0[i, pl.ds(f * L, L)] = jnp.zeros((L,), jnp.float32)
        return 0
    lax.fori_loop(0, CH, zb, 0)
    for k in range(ACC_ROWS // NS // ZCH):
        pltpu.sync_copy(rows0, acc.at[pl.ds(row0 + k * ZCH, ZCH)])

    def scale(buf, j):
        def grp(g, _):
            wvec = wv[j, pl.ds(g * L, L)]
            for l in range(L):
                w = wvec[l]
                i = g * L + l
                for f in range(D // L):
                    sl = pl.ds(f * L, L)
                    buf[i, sl] = buf[i, sl] * w
            return 0
        lax.fori_loop(0, CH // L, grp, 0)

    def run_phase(sref, dref, wref, nblocks):
        bufs = (rows0, rows1)
        gsems = (gsem0, gsem1)
        ssems = (ssem0, ssem1)

        def blk_body(b, _):
            pltpu.sync_copy(sref.at[cid, sid, pl.ds(b * BLK, BLK)], srcv)
            pltpu.sync_copy(dref.at[cid, sid, pl.ds(b * BLK, BLK)], dstv)
            pltpu.sync_copy(wref.at[cid, sid, pl.ds(b * BLK, BLK)], wv)

            gd = [None] * BLK
            sd = [None] * BLK
            gd[0] = pltpu.async_copy(pre_hbm.at[srcv.at[0]], bufs[0],
                                     gsems[0])
            for k in range(BLK):
                p = k % 2
                gd[k].wait()
                if k + 1 < BLK:
                    if k >= 1:
                        sd[k - 1].wait()
                    gd[k + 1] = pltpu.async_copy(
                        pre_hbm.at[srcv.at[k + 1]], bufs[1 - p],
                        gsems[1 - p])
                scale(bufs[p], k)
                sd[k] = pltpu.async_copy(bufs[p], acc.at[dstv.at[k]],
                                         ssems[p], add=True)
            sd[BLK - 2].wait()
            sd[BLK - 1].wait()
            return 0
        lax.fori_loop(0, nblocks, blk_body, 0)

    plsc.subcore_barrier()
    run_phase(s1, d1, w1, C1 // BLK)
    plsc.subcore_barrier()

    for k in range(ACC_ROWS // NS // ZCH):
        r = row0 + k * ZCH
        pltpu.sync_copy(acc.at[pl.ds(r, ZCH)], rows0)
        pltpu.sync_copy(rows0, pmain.at[cid, pl.ds(r, ZCH)])

    plsc.subcore_barrier()
    run_phase(s2, d2, w2, C2 // BLK)
    plsc.subcore_barrier()

    for k in range(ACC_ROWS // NS // ZCH):
        r = row0 + k * ZCH
        pltpu.sync_copy(acc.at[pl.ds(r, ZCH)], rows0)
        pltpu.sync_copy(rows0, pout.at[cid, pl.ds(r, ZCH)])


_spmm = functools.partial(
    pl.kernel,
    out_type=(jax.ShapeDtypeStruct((NC, ACC_ROWS, D), jnp.float32),
              jax.ShapeDtypeStruct((NC, ACC_ROWS, D), jnp.float32)),
    mesh=plsc.VectorSubcoreMesh(core_axis_name="c", subcore_axis_name="s"),
    scratch_types=[
        pltpu.VMEM((BLK, CH), jnp.int32),
        pltpu.VMEM((BLK, CH), jnp.int32),
        pltpu.VMEM((BLK, CH), jnp.float32),
        pltpu.VMEM((CH, D), jnp.float32),
        pltpu.VMEM((CH, D), jnp.float32),
        pltpu.VMEM_SHARED((ACC_ROWS, D), jnp.float32),
        pltpu.SemaphoreType.DMA,
        pltpu.SemaphoreType.DMA,
        pltpu.SemaphoreType.DMA,
        pltpu.SemaphoreType.DMA,
    ],
)(_spmm_body)


def _dense_body(pre_ref, a0_ref, a1_ref, p0_ref, p1_ref, q_ref, w1_ref,
                w2_ref, b_ref, m_ref, o_ref):
    hi = lax.Precision.HIGHEST
    agg = a0_ref[0] + a1_ref[0]
    pair = (p0_ref[0] - a0_ref[0]) + (p1_ref[0] - a1_ref[0])
    h = jnp.tanh(lax.dot(agg, q_ref[0], precision=hi))
    z = (lax.dot(pre_ref[...], w1_ref[0], precision=hi)
         + lax.dot(h, w2_ref[0], precision=hi) + b_ref[0])
    z = jnp.where(z >= 0, z, 0.01 * z)
    z = z / (jnp.sqrt(jnp.sum(z * z, axis=1, keepdims=True)) + 1e-12)
    o_ref[...] = z + jnp.tanh(lax.dot(pair, m_ref[0], precision=hi))


def _dense(pre, pmain, pout, qs, w1s, w2s, bs, ms, blk, nub):
    n = pre.shape[0]
    grid = n // blk
    row = pl.BlockSpec((blk, D), lambda i: (i, 0))
    part0 = pl.BlockSpec((1, blk, D), lambda i: (0, i, 0))
    part1 = pl.BlockSpec((1, blk, D), lambda i: (1, i, 0))
    wsel = pl.BlockSpec((1, D, D), lambda i: (i // nub, 0, 0))
    bsel = pl.BlockSpec((1, 1, D), lambda i: (i // nub, 0, 0))
    return pl.pallas_call(
        _dense_body,
        grid=(grid,),
        in_specs=[row, part0, part1, part0, part1, wsel, wsel, wsel,
                  bsel, wsel],
        out_specs=row,
        out_shape=jax.ShapeDtypeStruct((n, D), jnp.float32),
    )(pre, pmain, pmain, pout, pout, qs, w1s, w2s, bs, ms)


def _pad_to(x, total):
    return jnp.concatenate(
        [x, jnp.zeros((total - x.shape[0],), dtype=x.dtype)])


def _pad_idx(x, total, base, span):
    npad = total - x.shape[0]
    pad = base + (jnp.arange(npad, dtype=x.dtype) % span)
    return jnp.concatenate([x, pad])


def kernel(edge_index, edge_weight, sym_pair_edge_index, sym_pair_weight,
           herb_pair_edge_index, herb_pair_weight, user_embedding,
           item_embedding, Q_user_0, W_gc_user_0, b_gc_user_0, Q_item_0,
           W_gc_item_0, b_gc_item_0, M_user, M_item):
    nu = user_embedding.shape[0]
    ni = item_embedding.shape[0]
    n = nu + ni
    pre = jnp.concatenate([user_embedding, item_embedding], axis=0)

    t1 = NC * NS * C1 * CH
    s1 = _pad_idx(edge_index[0], t1, 0, n).reshape(NC, NS, C1, CH)
    d1 = _pad_idx(edge_index[1], t1, n, ACC_ROWS - n).reshape(NC, NS, C1, CH)
    w1 = _pad_to(edge_weight, t1).reshape(NC, NS, C1, CH)

    t2 = NC * NS * C2 * CH
    ps = jnp.concatenate([sym_pair_edge_index[0],
                          herb_pair_edge_index[0] + nu])
    pd = jnp.concatenate([sym_pair_edge_index[1],
                          herb_pair_edge_index[1] + PAIR_OFF])
    pw = jnp.concatenate([sym_pair_weight, herb_pair_weight])
    s2 = _pad_idx(ps, t2, 0, n).reshape(NC, NS, C2, CH)
    d2 = _pad_idx(pd, t2, PAIR_OFF + ni,
                  ACC_ROWS - PAIR_OFF - ni).reshape(NC, NS, C2, CH)
    w2 = _pad_to(pw, t2).reshape(NC, NS, C2, CH)

    pmain, pout = _spmm(pre, s1, d1, w1, s2, d2, w2)

    qs = jnp.stack([Q_user_0, Q_item_0])
    w1s = jnp.stack([W_gc_user_0[:D], W_gc_item_0[:D]])
    w2s = jnp.stack([W_gc_user_0[D:], W_gc_item_0[D:]])
    bs = jnp.stack([b_gc_user_0, b_gc_item_0])
    ms = jnp.stack([M_user, M_item])
    return _dense(pre, pmain, pout, qs, w1s, w2s, bs, ms, blk=1000,
                  nub=nu // 1000)


# speedup vs baseline: 1.1553x; 1.0204x over previous
import functools

import jax
import jax.numpy as jnp
from jax import lax
from jax.experimental import pallas as pl
from jax.experimental.pallas import tpu as pltpu
from jax.experimental.pallas import tpu_sc as plsc

NC, NS, L = 2, 16, 16
D = 128
CH = 128
BLK = 8
C1 = 80
C2 = 40
ACC_ROWS = 10240
PAIR_OFF = 6000
ZCH = 128


def _spmm_body(pre_hbm, e1, w1, e2, w2, pmain, pout,
               srcv, dstv, wv, rows0, rows1, acc,
               gsem0, gsem1, ssem0, ssem1):
    cid = lax.axis_index("c")
    sid = lax.axis_index("s")
    row0 = sid * (ACC_ROWS // NS)

    def zb(i, _):
        for f in range(D // L):
            rows0[i, pl.ds(f * L, L)] = jnp.zeros((L,), jnp.float32)
        return 0
    lax.fori_loop(0, CH, zb, 0)
    for k in range(ACC_ROWS // NS // ZCH):
        pltpu.sync_copy(rows0, acc.at[pl.ds(row0 + k * ZCH, ZCH)])

    def scale(buf, j):
        def grp(g, _):
            wvec = wv[j, pl.ds(g * L, L)]
            for l in range(L):
                w = wvec[l]
                i = g * L + l
                for f in range(D // L):
                    sl = pl.ds(f * L, L)
                    buf[i, sl] = buf[i, sl] * w
            return 0
        lax.fori_loop(0, CH // L, grp, 0)

    def run_phase(eref, wref, nblocks):
        bufs = (rows0, rows1)
        gsems = (gsem0, gsem1)
        ssems = (ssem0, ssem1)

        def blk_body(b, _):
            pltpu.sync_copy(eref.at[0, cid, sid, pl.ds(b * BLK, BLK)], srcv)
            pltpu.sync_copy(eref.at[1, cid, sid, pl.ds(b * BLK, BLK)], dstv)
            pltpu.sync_copy(wref.at[cid, sid, pl.ds(b * BLK, BLK)], wv)

            gd = [None] * BLK
            sd = [None] * BLK
            gd[0] = pltpu.async_copy(pre_hbm.at[srcv.at[0]], bufs[0],
                                     gsems[0])
            for k in range(BLK):
                p = k % 2
                gd[k].wait()
                if k + 1 < BLK:
                    if k >= 1:
                        sd[k - 1].wait()
                    gd[k + 1] = pltpu.async_copy(
                        pre_hbm.at[srcv.at[k + 1]], bufs[1 - p],
                        gsems[1 - p])
                scale(bufs[p], k)
                sd[k] = pltpu.async_copy(bufs[p], acc.at[dstv.at[k]],
                                         ssems[p], add=True)
            sd[BLK - 2].wait()
            sd[BLK - 1].wait()
            return 0
        lax.fori_loop(0, nblocks, blk_body, 0)

    plsc.subcore_barrier()
    run_phase(e1, w1, C1 // BLK)
    plsc.subcore_barrier()

    for k in range(ACC_ROWS // NS // ZCH):
        r = row0 + k * ZCH
        pltpu.sync_copy(acc.at[pl.ds(r, ZCH)], rows0)
        pltpu.sync_copy(rows0, pmain.at[cid, pl.ds(r, ZCH)])

    plsc.subcore_barrier()
    run_phase(e2, w2, C2 // BLK)
    plsc.subcore_barrier()

    for k in range(ACC_ROWS // NS // ZCH):
        r = row0 + k * ZCH
        pltpu.sync_copy(acc.at[pl.ds(r, ZCH)], rows0)
        pltpu.sync_copy(rows0, pout.at[cid, pl.ds(r, ZCH)])


_spmm = functools.partial(
    pl.kernel,
    out_type=(jax.ShapeDtypeStruct((NC, ACC_ROWS, D), jnp.float32),
              jax.ShapeDtypeStruct((NC, ACC_ROWS, D), jnp.float32)),
    mesh=plsc.VectorSubcoreMesh(core_axis_name="c", subcore_axis_name="s"),
    scratch_types=[
        pltpu.VMEM((BLK, CH), jnp.int32),
        pltpu.VMEM((BLK, CH), jnp.int32),
        pltpu.VMEM((BLK, CH), jnp.float32),
        pltpu.VMEM((CH, D), jnp.float32),
        pltpu.VMEM((CH, D), jnp.float32),
        pltpu.VMEM_SHARED((ACC_ROWS, D), jnp.float32),
        pltpu.SemaphoreType.DMA,
        pltpu.SemaphoreType.DMA,
        pltpu.SemaphoreType.DMA,
        pltpu.SemaphoreType.DMA,
    ],
)(_spmm_body)


def _dense_body(pre_ref, a0_ref, a1_ref, p0_ref, p1_ref, q_ref, w1_ref,
                w2_ref, b_ref, m_ref, o_ref):
    hi = lax.Precision.HIGHEST
    agg = a0_ref[0] + a1_ref[0]
    pair = (p0_ref[0] - a0_ref[0]) + (p1_ref[0] - a1_ref[0])
    h = jnp.tanh(lax.dot(agg, q_ref[0], precision=hi))
    z = (lax.dot(pre_ref[...], w1_ref[0], precision=hi)
         + lax.dot(h, w2_ref[0], precision=hi) + b_ref[0])
    z = jnp.where(z >= 0, z, 0.01 * z)
    z = z / (jnp.sqrt(jnp.sum(z * z, axis=1, keepdims=True)) + 1e-12)
    o_ref[...] = z + jnp.tanh(lax.dot(pair, m_ref[0], precision=hi))


def _dense(pre, pmain, pout, qs, w1s, w2s, bs, ms, blk, nub):
    n = pre.shape[0]
    grid = n // blk
    row = pl.BlockSpec((blk, D), lambda i: (i, 0))
    part0 = pl.BlockSpec((1, blk, D), lambda i: (0, i, 0))
    part1 = pl.BlockSpec((1, blk, D), lambda i: (1, i, 0))
    wsel = pl.BlockSpec((1, D, D), lambda i: (i // nub, 0, 0))
    bsel = pl.BlockSpec((1, 1, D), lambda i: (i // nub, 0, 0))
    return pl.pallas_call(
        _dense_body,
        grid=(grid,),
        in_specs=[row, part0, part1, part0, part1, wsel, wsel, wsel,
                  bsel, wsel],
        out_specs=row,
        out_shape=jax.ShapeDtypeStruct((n, D), jnp.float32),
    )(pre, pmain, pmain, pout, pout, qs, w1s, w2s, bs, ms)


def _pad_to(x, total):
    return jnp.concatenate(
        [x, jnp.zeros((total - x.shape[0],), dtype=x.dtype)])


def _pad_idx(x, total, base, span):
    npad = total - x.shape[0]
    pad = base + (jnp.arange(npad, dtype=x.dtype) % span)
    return jnp.concatenate([x, pad])


def kernel(edge_index, edge_weight, sym_pair_edge_index, sym_pair_weight,
           herb_pair_edge_index, herb_pair_weight, user_embedding,
           item_embedding, Q_user_0, W_gc_user_0, b_gc_user_0, Q_item_0,
           W_gc_item_0, b_gc_item_0, M_user, M_item):
    nu = user_embedding.shape[0]
    ni = item_embedding.shape[0]
    n = nu + ni
    pre = jnp.concatenate([user_embedding, item_embedding], axis=0)

    t1 = NC * NS * C1 * CH
    np1 = t1 - edge_index.shape[1]
    pad1 = jnp.stack([jnp.arange(np1, dtype=jnp.int32) % n,
                      n + jnp.arange(np1, dtype=jnp.int32) % (ACC_ROWS - n)])
    e1 = jnp.concatenate([edge_index, pad1], axis=1).reshape(
        2, NC, NS, C1, CH)
    w1 = _pad_to(edge_weight, t1).reshape(NC, NS, C1, CH)

    t2 = NC * NS * C2 * CH
    np2 = t2 - sym_pair_edge_index.shape[1] - herb_pair_edge_index.shape[1]
    ar2 = jnp.arange(np2, dtype=jnp.int32)
    ps = jnp.concatenate([sym_pair_edge_index[0],
                          herb_pair_edge_index[0] + nu, ar2 % n])
    pd = jnp.concatenate([sym_pair_edge_index[1],
                          herb_pair_edge_index[1] + PAIR_OFF,
                          PAIR_OFF + ni + ar2 % (ACC_ROWS - PAIR_OFF - ni)])
    e2 = jnp.stack([ps, pd]).reshape(2, NC, NS, C2, CH)
    w2 = _pad_to(jnp.concatenate([sym_pair_weight, herb_pair_weight]),
                 t2).reshape(NC, NS, C2, CH)

    pmain, pout = _spmm(pre, e1, w1, e2, w2)

    qs = jnp.stack([Q_user_0, Q_item_0])
    w1s = jnp.stack([W_gc_user_0[:D], W_gc_item_0[:D]])
    w2s = jnp.stack([W_gc_user_0[D:], W_gc_item_0[D:]])
    bs = jnp.stack([b_gc_user_0, b_gc_item_0])
    ms = jnp.stack([M_user, M_item])
    return _dense(pre, pmain, pout, qs, w1s, w2s, bs, ms, blk=1000,
                  nub=nu // 1000)


# speedup vs baseline: 1.2161x; 1.0527x over previous
import functools

import jax
import jax.numpy as jnp
from jax import lax
from jax.experimental import pallas as pl
from jax.experimental.pallas import tpu as pltpu
from jax.experimental.pallas import tpu_sc as plsc

NC, NS, L = 2, 16, 16
D = 128
CH = 128
BLK = 8
C1 = 80
C2 = 40
ACC_ROWS = 10240
PAIR_OFF = 6000
ZCH = 128


def _spmm_body(pre_hbm, e1, w1, e2, w2, pmain, pout,
               srcv, dstv, wv, rows0, rows1, acc,
               gsem0, gsem1, ssem0, ssem1, stsem):
    cid = lax.axis_index("c")
    sid = lax.axis_index("s")
    row0 = sid * (ACC_ROWS // NS)

    def zb(i, _):
        for f in range(D // L):
            rows0[i, pl.ds(f * L, L)] = jnp.zeros((L,), jnp.float32)
        return 0
    lax.fori_loop(0, CH, zb, 0)
    for k in range(ACC_ROWS // NS // ZCH):
        pltpu.sync_copy(rows0, acc.at[pl.ds(row0 + k * ZCH, ZCH)])

    def scale(buf, j):
        def grp(g, _):
            wvec = wv[j, pl.ds(g * L, L)]
            for l in range(L):
                w = wvec[l]
                i = g * L + l
                for f in range(D // L):
                    sl = pl.ds(f * L, L)
                    buf[i, sl] = buf[i, sl] * w
            return 0
        lax.fori_loop(0, CH // L, grp, 0)

    def run_phase(eref, wref, nblocks):
        bufs = (rows0, rows1)
        gsems = (gsem0, gsem1)
        ssems = (ssem0, ssem1)

        def blk_body(b, _):
            sl = pl.ds(b * BLK, BLK)
            st0 = pltpu.async_copy(eref.at[0, cid, sid, sl], srcv, stsem)
            st1 = pltpu.async_copy(eref.at[1, cid, sid, sl], dstv, stsem)
            st2 = pltpu.async_copy(wref.at[cid, sid, sl], wv, stsem)
            st0.wait()
            st1.wait()
            st2.wait()

            gd = [None] * BLK
            sd = [None] * BLK
            gd[0] = pltpu.async_copy(pre_hbm.at[srcv.at[0]], bufs[0],
                                     gsems[0])
            for k in range(BLK):
                p = k % 2
                gd[k].wait()
                if k + 1 < BLK:
                    if k >= 1:
                        sd[k - 1].wait()
                    gd[k + 1] = pltpu.async_copy(
                        pre_hbm.at[srcv.at[k + 1]], bufs[1 - p],
                        gsems[1 - p])
                scale(bufs[p], k)
                sd[k] = pltpu.async_copy(bufs[p], acc.at[dstv.at[k]],
                                         ssems[p], add=True)
            sd[BLK - 2].wait()
            sd[BLK - 1].wait()
            return 0
        lax.fori_loop(0, nblocks, blk_body, 0)

    plsc.subcore_barrier()
    run_phase(e1, w1, C1 // BLK)
    plsc.subcore_barrier()

    for k in range(ACC_ROWS // NS // ZCH):
        r = row0 + k * ZCH
        pltpu.sync_copy(acc.at[pl.ds(r, ZCH)], rows0)
        pltpu.sync_copy(rows0, pmain.at[cid, pl.ds(r, ZCH)])

    plsc.subcore_barrier()
    run_phase(e2, w2, C2 // BLK)
    plsc.subcore_barrier()

    for k in range(ACC_ROWS // NS // ZCH):
        r = row0 + k * ZCH
        pltpu.sync_copy(acc.at[pl.ds(r, ZCH)], rows0)
        pltpu.sync_copy(rows0, pout.at[cid, pl.ds(r, ZCH)])


_spmm = functools.partial(
    pl.kernel,
    out_type=(jax.ShapeDtypeStruct((NC, ACC_ROWS, D), jnp.float32),
              jax.ShapeDtypeStruct((NC, ACC_ROWS, D), jnp.float32)),
    mesh=plsc.VectorSubcoreMesh(core_axis_name="c", subcore_axis_name="s"),
    scratch_types=[
        pltpu.VMEM((BLK, CH), jnp.int32),
        pltpu.VMEM((BLK, CH), jnp.int32),
        pltpu.VMEM((BLK, CH), jnp.float32),
        pltpu.VMEM((CH, D), jnp.float32),
        pltpu.VMEM((CH, D), jnp.float32),
        pltpu.VMEM_SHARED((ACC_ROWS, D), jnp.float32),
        pltpu.SemaphoreType.DMA,
        pltpu.SemaphoreType.DMA,
        pltpu.SemaphoreType.DMA,
        pltpu.SemaphoreType.DMA,
        pltpu.SemaphoreType.DMA,
    ],
)(_spmm_body)


def _dense_body(pre_ref, a0_ref, a1_ref, p0_ref, p1_ref, q_ref, w1_ref,
                w2_ref, b_ref, m_ref, o_ref):
    hi = lax.Precision.HIGHEST
    agg = a0_ref[0] + a1_ref[0]
    pair = (p0_ref[0] - a0_ref[0]) + (p1_ref[0] - a1_ref[0])
    h = jnp.tanh(lax.dot(agg, q_ref[0], precision=hi))
    z = (lax.dot(pre_ref[...], w1_ref[0], precision=hi)
         + lax.dot(h, w2_ref[0], precision=hi) + b_ref[0])
    z = jnp.where(z >= 0, z, 0.01 * z)
    z = z / (jnp.sqrt(jnp.sum(z * z, axis=1, keepdims=True)) + 1e-12)
    o_ref[...] = z + jnp.tanh(lax.dot(pair, m_ref[0], precision=hi))


def _dense(pre, pmain, pout, qs, w1s, w2s, bs, ms, blk, nub):
    n = pre.shape[0]
    grid = n // blk
    row = pl.BlockSpec((blk, D), lambda i: (i, 0))
    part0 = pl.BlockSpec((1, blk, D), lambda i: (0, i, 0))
    part1 = pl.BlockSpec((1, blk, D), lambda i: (1, i, 0))
    wsel = pl.BlockSpec((1, D, D), lambda i: (i // nub, 0, 0))
    bsel = pl.BlockSpec((1, 1, D), lambda i: (i // nub, 0, 0))
    return pl.pallas_call(
        _dense_body,
        grid=(grid,),
        in_specs=[row, part0, part1, part0, part1, wsel, wsel, wsel,
                  bsel, wsel],
        out_specs=row,
        out_shape=jax.ShapeDtypeStruct((n, D), jnp.float32),
    )(pre, pmain, pmain, pout, pout, qs, w1s, w2s, bs, ms)


def _pad_to(x, total):
    return jnp.concatenate(
        [x, jnp.zeros((total - x.shape[0],), dtype=x.dtype)])


def _pad_idx(x, total, base, span):
    npad = total - x.shape[0]
    pad = base + (jnp.arange(npad, dtype=x.dtype) % span)
    return jnp.concatenate([x, pad])


def kernel(edge_index, edge_weight, sym_pair_edge_index, sym_pair_weight,
           herb_pair_edge_index, herb_pair_weight, user_embedding,
           item_embedding, Q_user_0, W_gc_user_0, b_gc_user_0, Q_item_0,
           W_gc_item_0, b_gc_item_0, M_user, M_item):
    nu = user_embedding.shape[0]
    ni = item_embedding.shape[0]
    n = nu + ni
    pre = jnp.concatenate([user_embedding, item_embedding], axis=0)

    t1 = NC * NS * C1 * CH
    np1 = t1 - edge_index.shape[1]
    pad1 = jnp.stack([jnp.arange(np1, dtype=jnp.int32) % n,
                      n + jnp.arange(np1, dtype=jnp.int32) % (ACC_ROWS - n)])
    e1 = jnp.concatenate([edge_index, pad1], axis=1).reshape(
        2, NC, NS, C1, CH)
    w1 = _pad_to(edge_weight, t1).reshape(NC, NS, C1, CH)

    t2 = NC * NS * C2 * CH
    np2 = t2 - sym_pair_edge_index.shape[1] - herb_pair_edge_index.shape[1]
    ar2 = jnp.arange(np2, dtype=jnp.int32)
    ps = jnp.concatenate([sym_pair_edge_index[0],
                          herb_pair_edge_index[0] + nu, ar2 % n])
    pd = jnp.concatenate([sym_pair_edge_index[1],
                          herb_pair_edge_index[1] + PAIR_OFF,
                          PAIR_OFF + ni + ar2 % (ACC_ROWS - PAIR_OFF - ni)])
    e2 = jnp.stack([ps, pd]).reshape(2, NC, NS, C2, CH)
    w2 = _pad_to(jnp.concatenate([sym_pair_weight, herb_pair_weight]),
                 t2).reshape(NC, NS, C2, CH)

    pmain, pout = _spmm(pre, e1, w1, e2, w2)

    qs = jnp.stack([Q_user_0, Q_item_0])
    w1s = jnp.stack([W_gc_user_0[:D], W_gc_item_0[:D]])
    w2s = jnp.stack([W_gc_user_0[D:], W_gc_item_0[D:]])
    bs = jnp.stack([b_gc_user_0, b_gc_item_0])
    ms = jnp.stack([M_user, M_item])
    return _dense(pre, pmain, pout, qs, w1s, w2s, bs, ms, blk=1000,
                  nub=nu // 1000)


# speedup vs baseline: 1.2248x; 1.0072x over previous
import functools

import jax
import jax.numpy as jnp
from jax import lax
from jax.experimental import pallas as pl
from jax.experimental.pallas import tpu as pltpu
from jax.experimental.pallas import tpu_sc as plsc

NC, NS, L = 2, 16, 16
D = 128
CH = 128
BLK = 8
C1 = 80
C2 = 40
ACC_ROWS = 10240
PAIR_OFF = 6000
ZCH = 128


def _spmm_body(pre_hbm, e1, w1, e2, w2, pmain, pout,
               srcv, dstv, wv, rows0, rows1, acc,
               gsem0, gsem1, ssem0, ssem1, stsem):
    cid = lax.axis_index("c")
    sid = lax.axis_index("s")
    row0 = sid * (ACC_ROWS // NS)

    def zb(i, _):
        for f in range(D // L):
            rows0[i, pl.ds(f * L, L)] = jnp.zeros((L,), jnp.float32)
        return 0
    lax.fori_loop(0, CH, zb, 0)
    zd = [pltpu.async_copy(rows0, acc.at[pl.ds(row0 + k * ZCH, ZCH)], stsem)
          for k in range(ACC_ROWS // NS // ZCH)]
    for d in zd:
        d.wait()

    def scale(buf, j):
        def grp(g, _):
            wvec = wv[j, pl.ds(g * L, L)]
            for l in range(L):
                w = wvec[l]
                i = g * L + l
                for f in range(D // L):
                    sl = pl.ds(f * L, L)
                    buf[i, sl] = buf[i, sl] * w
            return 0
        lax.fori_loop(0, CH // L, grp, 0)

    def run_phase(eref, wref, nblocks):
        bufs = (rows0, rows1)
        gsems = (gsem0, gsem1)
        ssems = (ssem0, ssem1)

        def blk_body(b, _):
            sl = pl.ds(b * BLK, BLK)
            st0 = pltpu.async_copy(eref.at[0, cid, sid, sl], srcv, stsem)
            st1 = pltpu.async_copy(eref.at[1, cid, sid, sl], dstv, stsem)
            st2 = pltpu.async_copy(wref.at[cid, sid, sl], wv, stsem)
            st0.wait()
            st1.wait()
            st2.wait()

            gd = [None] * BLK
            sd = [None] * BLK
            gd[0] = pltpu.async_copy(pre_hbm.at[srcv.at[0]], bufs[0],
                                     gsems[0])
            for k in range(BLK):
                p = k % 2
                gd[k].wait()
                if k + 1 < BLK:
                    if k >= 1:
                        sd[k - 1].wait()
                    gd[k + 1] = pltpu.async_copy(
                        pre_hbm.at[srcv.at[k + 1]], bufs[1 - p],
                        gsems[1 - p])
                scale(bufs[p], k)
                sd[k] = pltpu.async_copy(bufs[p], acc.at[dstv.at[k]],
                                         ssems[p], add=True)
            sd[BLK - 2].wait()
            sd[BLK - 1].wait()
            return 0
        lax.fori_loop(0, nblocks, blk_body, 0)

    plsc.subcore_barrier()
    run_phase(e1, w1, C1 // BLK)
    plsc.subcore_barrier()

    def copy_out(dst):
        nk = ACC_ROWS // NS // ZCH
        bufs = (rows0, rows1)
        gsems = (gsem0, gsem1)
        ssems = (ssem0, ssem1)
        rd = [None] * nk
        wr = [None] * nk
        rd[0] = pltpu.async_copy(acc.at[pl.ds(row0, ZCH)], bufs[0], gsems[0])
        for k in range(nk):
            p = k % 2
            rd[k].wait()
            if k + 1 < nk:
                if k >= 1:
                    wr[k - 1].wait()
                rd[k + 1] = pltpu.async_copy(
                    acc.at[pl.ds(row0 + (k + 1) * ZCH, ZCH)], bufs[1 - p],
                    gsems[1 - p])
            wr[k] = pltpu.async_copy(
                bufs[p], dst.at[cid, pl.ds(row0 + k * ZCH, ZCH)], ssems[p])
        wr[nk - 2].wait()
        wr[nk - 1].wait()

    copy_out(pmain)

    plsc.subcore_barrier()
    run_phase(e2, w2, C2 // BLK)
    plsc.subcore_barrier()

    copy_out(pout)


_spmm = functools.partial(
    pl.kernel,
    out_type=(jax.ShapeDtypeStruct((NC, ACC_ROWS, D), jnp.float32),
              jax.ShapeDtypeStruct((NC, ACC_ROWS, D), jnp.float32)),
    mesh=plsc.VectorSubcoreMesh(core_axis_name="c", subcore_axis_name="s"),
    scratch_types=[
        pltpu.VMEM((BLK, CH), jnp.int32),
        pltpu.VMEM((BLK, CH), jnp.int32),
        pltpu.VMEM((BLK, CH), jnp.float32),
        pltpu.VMEM((CH, D), jnp.float32),
        pltpu.VMEM((CH, D), jnp.float32),
        pltpu.VMEM_SHARED((ACC_ROWS, D), jnp.float32),
        pltpu.SemaphoreType.DMA,
        pltpu.SemaphoreType.DMA,
        pltpu.SemaphoreType.DMA,
        pltpu.SemaphoreType.DMA,
        pltpu.SemaphoreType.DMA,
    ],
)(_spmm_body)


def _dense_body(pre_ref, a0_ref, a1_ref, p0_ref, p1_ref, q_ref, w1_ref,
                w2_ref, b_ref, m_ref, o_ref):
    hi = lax.Precision.HIGHEST
    agg = a0_ref[0] + a1_ref[0]
    pair = (p0_ref[0] - a0_ref[0]) + (p1_ref[0] - a1_ref[0])
    h = jnp.tanh(lax.dot(agg, q_ref[0], precision=hi))
    z = (lax.dot(pre_ref[...], w1_ref[0], precision=hi)
         + lax.dot(h, w2_ref[0], precision=hi) + b_ref[0])
    z = jnp.where(z >= 0, z, 0.01 * z)
    z = z / (jnp.sqrt(jnp.sum(z * z, axis=1, keepdims=True)) + 1e-12)
    o_ref[...] = z + jnp.tanh(lax.dot(pair, m_ref[0], precision=hi))


def _dense(pre, pmain, pout, qs, w1s, w2s, bs, ms, blk, nub):
    n = pre.shape[0]
    grid = n // blk
    row = pl.BlockSpec((blk, D), lambda i: (i, 0))
    part0 = pl.BlockSpec((1, blk, D), lambda i: (0, i, 0))
    part1 = pl.BlockSpec((1, blk, D), lambda i: (1, i, 0))
    wsel = pl.BlockSpec((1, D, D), lambda i: (i // nub, 0, 0))
    bsel = pl.BlockSpec((1, 1, D), lambda i: (i // nub, 0, 0))
    return pl.pallas_call(
        _dense_body,
        grid=(grid,),
        in_specs=[row, part0, part1, part0, part1, wsel, wsel, wsel,
                  bsel, wsel],
        out_specs=row,
        out_shape=jax.ShapeDtypeStruct((n, D), jnp.float32),
    )(pre, pmain, pmain, pout, pout, qs, w1s, w2s, bs, ms)


def _pad_to(x, total):
    return jnp.concatenate(
        [x, jnp.zeros((total - x.shape[0],), dtype=x.dtype)])


def _pad_idx(x, total, base, span):
    npad = total - x.shape[0]
    pad = base + (jnp.arange(npad, dtype=x.dtype) % span)
    return jnp.concatenate([x, pad])


def kernel(edge_index, edge_weight, sym_pair_edge_index, sym_pair_weight,
           herb_pair_edge_index, herb_pair_weight, user_embedding,
           item_embedding, Q_user_0, W_gc_user_0, b_gc_user_0, Q_item_0,
           W_gc_item_0, b_gc_item_0, M_user, M_item):
    nu = user_embedding.shape[0]
    ni = item_embedding.shape[0]
    n = nu + ni
    pre = jnp.concatenate([user_embedding, item_embedding], axis=0)

    t1 = NC * NS * C1 * CH
    np1 = t1 - edge_index.shape[1]
    pad1 = jnp.stack([jnp.arange(np1, dtype=jnp.int32) % n,
                      n + jnp.arange(np1, dtype=jnp.int32) % (ACC_ROWS - n)])
    e1 = jnp.concatenate([edge_index, pad1], axis=1).reshape(
        2, NC, NS, C1, CH)
    w1 = _pad_to(edge_weight, t1).reshape(NC, NS, C1, CH)

    t2 = NC * NS * C2 * CH
    np2 = t2 - sym_pair_edge_index.shape[1] - herb_pair_edge_index.shape[1]
    ar2 = jnp.arange(np2, dtype=jnp.int32)
    ps = jnp.concatenate([sym_pair_edge_index[0],
                          herb_pair_edge_index[0] + nu, ar2 % n])
    pd = jnp.concatenate([sym_pair_edge_index[1],
                          herb_pair_edge_index[1] + PAIR_OFF,
                          PAIR_OFF + ni + ar2 % (ACC_ROWS - PAIR_OFF - ni)])
    e2 = jnp.stack([ps, pd]).reshape(2, NC, NS, C2, CH)
    w2 = _pad_to(jnp.concatenate([sym_pair_weight, herb_pair_weight]),
                 t2).reshape(NC, NS, C2, CH)

    pmain, pout = _spmm(pre, e1, w1, e2, w2)

    qs = jnp.stack([Q_user_0, Q_item_0])
    w1s = jnp.stack([W_gc_user_0[:D], W_gc_item_0[:D]])
    w2s = jnp.stack([W_gc_user_0[D:], W_gc_item_0[D:]])
    bs = jnp.stack([b_gc_user_0, b_gc_item_0])
    ms = jnp.stack([M_user, M_item])
    return _dense(pre, pmain, pout, qs, w1s, w2s, bs, ms, blk=1000,
                  nub=nu // 1000)


# speedup vs baseline: 1.2276x; 1.0022x over previous
import functools

import jax
import jax.numpy as jnp
from jax import lax
from jax.experimental import pallas as pl
from jax.experimental.pallas import tpu as pltpu
from jax.experimental.pallas import tpu_sc as plsc

NC, NS, L = 2, 16, 16
D = 128
CH = 128
BLK = 8
C1 = 80
C2 = 40
ACC_ROWS = 10240
PAIR_OFF = 6000
ZCH = 128


def _spmm_body(pre_hbm, e1, w1, e2, w2, pmain, pout,
               srcv, dstv, wv, rows0, rows1, acc,
               gsem0, gsem1, ssem0, ssem1, stsem):
    cid = lax.axis_index("c")
    sid = lax.axis_index("s")
    row0 = sid * (ACC_ROWS // NS)

    def zb(i, _):
        for f in range(D // L):
            rows0[i, pl.ds(f * L, L)] = jnp.zeros((L,), jnp.float32)
        return 0
    lax.fori_loop(0, CH, zb, 0)
    zd = [pltpu.async_copy(rows0, acc.at[pl.ds(row0 + k * ZCH, ZCH)], stsem)
          for k in range(ACC_ROWS // NS // ZCH)]
    for d in zd:
        d.wait()

    def scale(buf, j):
        def grp(g, _):
            wvec = wv[j, pl.ds(g * L, L)]
            for l in range(L):
                w = wvec[l]
                i = g * L + l
                for f in range(D // L):
                    sl = pl.ds(f * L, L)
                    buf[i, sl] = buf[i, sl] * w
            return 0
        lax.fori_loop(0, CH // L, grp, 0)

    def run_phase(eref, wref, nblocks):
        bufs = (rows0, rows1)
        gsems = (gsem0, gsem1)
        ssems = (ssem0, ssem1)

        def blk_body(b, _):
            sl = pl.ds(b * BLK, BLK)
            st0 = pltpu.async_copy(eref.at[0, cid, sid, sl], srcv, stsem)
            st1 = pltpu.async_copy(eref.at[1, cid, sid, sl], dstv, stsem)
            st2 = pltpu.async_copy(wref.at[cid, sid, sl], wv, stsem)
            st0.wait()
            st1.wait()
            st2.wait()

            gd = [None] * BLK
            sd = [None] * BLK
            gd[0] = pltpu.async_copy(pre_hbm.at[srcv.at[0]], bufs[0],
                                     gsems[0])
            for k in range(BLK):
                p = k % 2
                gd[k].wait()
                if k + 1 < BLK:
                    if k >= 1:
                        sd[k - 1].wait()
                    gd[k + 1] = pltpu.async_copy(
                        pre_hbm.at[srcv.at[k + 1]], bufs[1 - p],
                        gsems[1 - p])
                scale(bufs[p], k)
                sd[k] = pltpu.async_copy(bufs[p], acc.at[dstv.at[k]],
                                         ssems[p], add=True)
            sd[BLK - 2].wait()
            sd[BLK - 1].wait()
            return 0
        lax.fori_loop(0, nblocks, blk_body, 0)

    plsc.subcore_barrier()
    run_phase(e1, w1, C1 // BLK)
    plsc.subcore_barrier()

    def copy_out(dst):
        nk = ACC_ROWS // NS // ZCH
        bufs = (rows0, rows1)
        gsems = (gsem0, gsem1)
        ssems = (ssem0, ssem1)
        rd = [None] * nk
        wr = [None] * nk
        rd[0] = pltpu.async_copy(acc.at[pl.ds(row0, ZCH)], bufs[0], gsems[0])
        for k in range(nk):
            p = k % 2
            rd[k].wait()
            if k + 1 < nk:
                if k >= 1:
                    wr[k - 1].wait()
                rd[k + 1] = pltpu.async_copy(
                    acc.at[pl.ds(row0 + (k + 1) * ZCH, ZCH)], bufs[1 - p],
                    gsems[1 - p])
            wr[k] = pltpu.async_copy(
                bufs[p], dst.at[cid, pl.ds(row0 + k * ZCH, ZCH)], ssems[p])
        wr[nk - 2].wait()
        wr[nk - 1].wait()

    copy_out(pmain)

    plsc.subcore_barrier()
    run_phase(e2, w2, C2 // BLK)
    plsc.subcore_barrier()

    copy_out(pout)


_spmm = functools.partial(
    pl.kernel,
    out_type=(jax.ShapeDtypeStruct((NC, ACC_ROWS, D), jnp.float32),
              jax.ShapeDtypeStruct((NC, ACC_ROWS, D), jnp.float32)),
    mesh=plsc.VectorSubcoreMesh(core_axis_name="c", subcore_axis_name="s"),
    scratch_types=[
        pltpu.VMEM((BLK, CH), jnp.int32),
        pltpu.VMEM((BLK, CH), jnp.int32),
        pltpu.VMEM((BLK, CH), jnp.float32),
        pltpu.VMEM((CH, D), jnp.float32),
        pltpu.VMEM((CH, D), jnp.float32),
        pltpu.VMEM_SHARED((ACC_ROWS, D), jnp.float32),
        pltpu.SemaphoreType.DMA,
        pltpu.SemaphoreType.DMA,
        pltpu.SemaphoreType.DMA,
        pltpu.SemaphoreType.DMA,
        pltpu.SemaphoreType.DMA,
    ],
)(_spmm_body)


def _dense_body(pre_ref, a0_ref, a1_ref, p0_ref, p1_ref, q_ref, w1_ref,
                w2_ref, b_ref, m_ref, o_ref):
    hi = lax.Precision.HIGHEST
    agg = a0_ref[0] + a1_ref[0]
    pair = (p0_ref[0] - a0_ref[0]) + (p1_ref[0] - a1_ref[0])
    h = jnp.tanh(lax.dot(agg, q_ref[0], precision=hi))
    z = (lax.dot(pre_ref[...], w1_ref[0], precision=hi)
         + lax.dot(h, w2_ref[0], precision=hi) + b_ref[0])
    z = jnp.where(z >= 0, z, 0.01 * z)
    z = z / (jnp.sqrt(jnp.sum(z * z, axis=1, keepdims=True)) + 1e-12)
    o_ref[...] = z + jnp.tanh(lax.dot(pair, m_ref[0], precision=hi))


def _dense(pre, pmain, pout, qs, w1s, w2s, bs, ms, blk, nub):
    n = pre.shape[0]
    grid = n // blk
    row = pl.BlockSpec((blk, D), lambda i: (i, 0))
    part0 = pl.BlockSpec((1, blk, D), lambda i: (0, i, 0))
    part1 = pl.BlockSpec((1, blk, D), lambda i: (1, i, 0))
    wsel = pl.BlockSpec((1, D, D), lambda i: (i // nub, 0, 0))
    bsel = pl.BlockSpec((1, 1, D), lambda i: (i // nub, 0, 0))
    return pl.pallas_call(
        _dense_body,
        grid=(grid,),
        in_specs=[row, part0, part1, part0, part1, wsel, wsel, wsel,
                  bsel, wsel],
        out_specs=row,
        out_shape=jax.ShapeDtypeStruct((n, D), jnp.float32),
    )(pre, pmain, pmain, pout, pout, qs, w1s, w2s, bs, ms)


def _pad_to(x, total):
    return jnp.concatenate(
        [x, jnp.zeros((total - x.shape[0],), dtype=x.dtype)])


def kernel(edge_index, edge_weight, sym_pair_edge_index, sym_pair_weight,
           herb_pair_edge_index, herb_pair_weight, user_embedding,
           item_embedding, Q_user_0, W_gc_user_0, b_gc_user_0, Q_item_0,
           W_gc_item_0, b_gc_item_0, M_user, M_item):
    nu = user_embedding.shape[0]
    ni = item_embedding.shape[0]
    n = nu + ni
    pre = jnp.concatenate([user_embedding, item_embedding], axis=0)

    t1 = NC * NS * C1 * CH
    np1 = t1 - edge_index.shape[1]
    pad1 = jnp.stack([jnp.arange(np1, dtype=jnp.int32) % n,
                      n + jnp.arange(np1, dtype=jnp.int32) % (ACC_ROWS - n)])
    e1 = jnp.concatenate([edge_index, pad1], axis=1).reshape(
        2, NC, NS, C1, CH)
    w1 = _pad_to(edge_weight, t1).reshape(NC, NS, C1, CH)

    t2 = NC * NS * C2 * CH
    np2 = t2 - sym_pair_edge_index.shape[1] - herb_pair_edge_index.shape[1]
    ar2 = jnp.arange(np2, dtype=jnp.int32)
    ps = jnp.concatenate([sym_pair_edge_index[0],
                          herb_pair_edge_index[0] + nu, ar2 % n])
    pd = jnp.concatenate([sym_pair_edge_index[1],
                          herb_pair_edge_index[1] + PAIR_OFF,
                          PAIR_OFF + ni + ar2 % (ACC_ROWS - PAIR_OFF - ni)])
    e2 = jnp.stack([ps, pd]).reshape(2, NC, NS, C2, CH)
    w2 = _pad_to(jnp.concatenate([sym_pair_weight, herb_pair_weight]),
                 t2).reshape(NC, NS, C2, CH)

    pmain, pout = _spmm(pre, e1, w1, e2, w2)

    qs = jnp.stack([Q_user_0, Q_item_0])
    w1s = jnp.stack([W_gc_user_0[:D], W_gc_item_0[:D]])
    w2s = jnp.stack([W_gc_user_0[D:], W_gc_item_0[D:]])
    bs = jnp.stack([b_gc_user_0, b_gc_item_0])
    ms = jnp.stack([M_user, M_item])
    return _dense(pre, pmain, pout, qs, w1s, w2s, bs, ms, blk=1000,
                  nub=nu // 1000)
